# baseline (speedup 1.0000x reference)
import functools

import jax
import jax.numpy as jnp
from jax import lax
from jax.experimental import pallas as pl
from jax.experimental.pallas import tpu as pltpu
from jax.experimental.pallas import tpu_sc as plsc

SEQ_LEN = 8192
MODEL_DIM = 1024
NUM_CORES = 2
NUM_SUBCORES = 16
NUM_WORKERS = NUM_CORES * NUM_SUBCORES
ROWS_PER_WORKER = SEQ_LEN // NUM_WORKERS
CHUNK_ROWS = 32
NUM_CHUNKS = ROWS_PER_WORKER // CHUNK_ROWS
NBUF = 3

_mesh = plsc.VectorSubcoreMesh(core_axis_name="c", subcore_axis_name="s")


@functools.partial(
    pl.kernel,
    mesh=_mesh,
    out_type=jax.ShapeDtypeStruct((SEQ_LEN, MODEL_DIM), jnp.float32),
    scratch_types=(
        [pltpu.VMEM_SHARED((NUM_SUBCORES, NBUF, CHUNK_ROWS, MODEL_DIM), jnp.float32)]
        + [pltpu.SemaphoreType.DMA] * (2 * NBUF)
    ),
)
def _identity_gather(emb_hbm, out_hbm, shared, *sems):
    in_sems = sems[:NBUF]
    out_sems = sems[NBUF:]
    cid = lax.axis_index("c")
    sid = lax.axis_index("s")
    wid = sid * NUM_CORES + cid
    base = wid * ROWS_PER_WORKER
    buf = shared.at[sid]

    def chunk_slice(i):
        return pl.ds(base + i * CHUNK_ROWS, CHUNK_ROWS)

    for i in range(NBUF - 1):
        pltpu.async_copy(emb_hbm.at[chunk_slice(i)], buf.at[i], in_sems[i])
    for i in range(NUM_CHUNKS):
        cur = i % NBUF
        j = i + NBUF - 1
        if j < NUM_CHUNKS:
            b = j % NBUF
            if j >= NBUF:
                pltpu.make_async_copy(
                    buf.at[b], out_hbm.at[chunk_slice(j - NBUF)], out_sems[b]
                ).wait()
            pltpu.async_copy(emb_hbm.at[chunk_slice(j)], buf.at[b], in_sems[b])
        pltpu.make_async_copy(
            emb_hbm.at[chunk_slice(i)], buf.at[cur], in_sems[cur]
        ).wait()
        pltpu.async_copy(buf.at[cur], out_hbm.at[chunk_slice(i)], out_sems[cur])
    for i in range(max(0, NUM_CHUNKS - NBUF), NUM_CHUNKS):
        cur = i % NBUF
        pltpu.make_async_copy(
            buf.at[cur], out_hbm.at[chunk_slice(i)], out_sems[cur]
        ).wait()


def kernel(x, emb):
    del x
    return _identity_gather(emb)


# speedup vs baseline: 1.0006x; 1.0006x over previous
import functools

import jax
import jax.numpy as jnp
from jax import lax
from jax.experimental import pallas as pl
from jax.experimental.pallas import tpu as pltpu
from jax.experimental.pallas import tpu_sc as plsc

SEQ_LEN = 8192
MODEL_DIM = 1024
NUM_CORES = 2
NUM_SUBCORES = 16
NUM_WORKERS = NUM_CORES * NUM_SUBCORES
ROWS_PER_WORKER = SEQ_LEN // NUM_WORKERS
CHUNK_ROWS = 16
NUM_CHUNKS = ROWS_PER_WORKER // CHUNK_ROWS
NCH = NUM_CHUNKS // 2
NBUF_A = 3
NBUF_B = 2

_mesh = plsc.VectorSubcoreMesh(core_axis_name="c", subcore_axis_name="s")


@functools.partial(
    pl.kernel,
    mesh=_mesh,
    out_type=jax.ShapeDtypeStruct((SEQ_LEN, MODEL_DIM), jnp.float32),
    scratch_types=(
        [
            pltpu.VMEM((NBUF_A, CHUNK_ROWS, MODEL_DIM), jnp.float32),
            pltpu.VMEM_SHARED(
                (NUM_SUBCORES, NBUF_B, CHUNK_ROWS, MODEL_DIM), jnp.float32
            ),
        ]
        + [pltpu.SemaphoreType.DMA] * (2 * NBUF_A + 2 * NBUF_B)
    ),
)
def _identity_gather(emb_hbm, out_hbm, tbuf, shared, *sems):
    a_in = sems[:NBUF_A]
    a_out = sems[NBUF_A : 2 * NBUF_A]
    b_in = sems[2 * NBUF_A : 2 * NBUF_A + NBUF_B]
    b_out = sems[2 * NBUF_A + NBUF_B :]
    cid = lax.axis_index("c")
    sid = lax.axis_index("s")
    wid = sid * NUM_CORES + cid
    base = wid * ROWS_PER_WORKER
    sbuf = shared.at[sid]

    def a_slice(i):
        return pl.ds(base + (2 * i) * CHUNK_ROWS, CHUNK_ROWS)

    def b_slice(i):
        return pl.ds(base + (2 * i + 1) * CHUNK_ROWS, CHUNK_ROWS)

    for i in range(NBUF_A - 1):
        pltpu.async_copy(emb_hbm.at[a_slice(i)], tbuf.at[i], a_in[i])
    for i in range(NBUF_B - 1):
        pltpu.async_copy(emb_hbm.at[b_slice(i)], sbuf.at[i], b_in[i])
    for i in range(NCH):
        j = i + NBUF_A - 1
        if j < NCH:
            b = j % NBUF_A
            if j >= NBUF_A:
                pltpu.make_async_copy(
                    tbuf.at[b], out_hbm.at[a_slice(j - NBUF_A)], a_out[b]
                ).wait()
            pltpu.async_copy(emb_hbm.at[a_slice(j)], tbuf.at[b], a_in[b])
        cur = i % NBUF_A
        pltpu.make_async_copy(emb_hbm.at[a_slice(i)], tbuf.at[cur], a_in[cur]).wait()
        pltpu.async_copy(tbuf.at[cur], out_hbm.at[a_slice(i)], a_out[cur])
        j = i + NBUF_B - 1
        if j < NCH:
            b = j % NBUF_B
            if j >= NBUF_B:
                pltpu.make_async_copy(
                    sbuf.at[b], out_hbm.at[b_slice(j - NBUF_B)], b_out[b]
                ).wait()
            pltpu.async_copy(emb_hbm.at[b_slice(j)], sbuf.at[b], b_in[b])
        cur = i % NBUF_B
        pltpu.make_async_copy(emb_hbm.at[b_slice(i)], sbuf.at[cur], b_in[cur]).wait()
        pltpu.async_copy(sbuf.at[cur], out_hbm.at[b_slice(i)], b_out[cur])
    for i in range(max(0, NCH - NBUF_A), NCH):
        cur = i % NBUF_A
        pltpu.make_async_copy(
            tbuf.at[cur], out_hbm.at[a_slice(i)], a_out[cur]
        ).wait()
    for i in range(max(0, NCH - NBUF_B), NCH):
        cur = i % NBUF_B
        pltpu.make_async_copy(
            sbuf.at[cur], out_hbm.at[b_slice(i)], b_out[cur]
        ).wait()


def kernel(x, emb):
    del x
    return _identity_gather(emb)
